# Initial kernel scaffold; baseline (speedup 1.0000x reference)
import jax
import jax.numpy as jnp
from jax.experimental import pallas as pl


def kernel(x, W):
    raise NotImplementedError("write your pallas kernel here")


# speedup vs baseline: 1.1996x; 1.1996x over previous
import functools

import jax
import jax.numpy as jnp
from jax.experimental import pallas as pl


def _router_body(x_ref, w_ref, probs_ref, logits_ref, z_ref, *, inv_n):
    i = pl.program_id(0)
    xb = x_ref[...]
    logits = jnp.dot(xb, w_ref[...],
                     preferred_element_type=jnp.float32)

    m = jnp.max(logits, axis=-1, keepdims=True)
    e = jnp.exp(logits - m)
    probs_ref[...] = e / jnp.sum(e, axis=-1, keepdims=True)

    absum = jnp.sum(jnp.abs(xb), axis=-1, keepdims=True)
    masked = jnp.where(absum > 0, logits, 0.0)
    logits_ref[...] = masked

    mm = jnp.max(masked, axis=-1, keepdims=True)
    lse = jnp.log(jnp.sum(jnp.exp(masked - mm), axis=-1, keepdims=True)) + mm
    part = jnp.sum(lse * lse) * inv_n

    @pl.when(i == 0)
    def _():
        z_ref[...] = jnp.zeros_like(z_ref)

    z_ref[...] = z_ref[...] + part


def kernel(x, W):
    b, s, d = x.shape
    n = b * s
    e = W.shape[1]
    xf = x.reshape(n, d)

    blk = 512
    body = functools.partial(_router_body, inv_n=1.0 / n)
    probs, logits, z = pl.pallas_call(
        body,
        grid=(n // blk,),
        in_specs=[
            pl.BlockSpec((blk, d), lambda i: (i, 0)),
            pl.BlockSpec((d, e), lambda i: (0, 0)),
        ],
        out_specs=[
            pl.BlockSpec((blk, e), lambda i: (i, 0)),
            pl.BlockSpec((blk, e), lambda i: (i, 0)),
            pl.BlockSpec((1, 1), lambda i: (0, 0)),
        ],
        out_shape=[
            jax.ShapeDtypeStruct((n, e), jnp.float32),
            jax.ShapeDtypeStruct((n, e), jnp.float32),
            jax.ShapeDtypeStruct((1, 1), jnp.float32),
        ],
    )(xf, W)
    return probs, logits, z[0, 0]
